# baseline (speedup 1.0000x reference)
import functools

import jax
import jax.numpy as jnp
from jax import lax
from jax.experimental import pallas as pl
from jax.experimental.pallas import tpu as pltpu
from jax.experimental.pallas import tpu_sc as plsc

_EMBED = 32
_P = 33
_NC = 2
_NS = 16
_NW = _NC * _NS
_VOCAB = 1000000
_VT_FULL = _VOCAB // 128
_TAIL = _VOCAB - _VT_FULL * 128
_NTW = 246
_L = 50
_B = 16384
_BW = _B // _NW

_MESH = plsc.VectorSubcoreMesh(core_axis_name="c", subcore_axis_name="s")


@jax.jit
def _detile_table(wt, tail_lin):
    @functools.partial(
        pl.kernel,
        mesh=_MESH,
        compiler_params=pltpu.CompilerParams(
            use_tc_tiling_on_sc=True, needs_layout_passes=False),
        out_type=jax.ShapeDtypeStruct((_VOCAB, _EMBED), jnp.float32),
        scratch_types=[
            pltpu.VMEM((_EMBED, 128), jnp.float32),
            pltpu.VMEM((_EMBED, 128), jnp.float32),
            pltpu.VMEM((128, _P), jnp.float32),
            pltpu.VMEM((128, _P), jnp.float32),
            pltpu.VMEM((128, _EMBED), jnp.float32),
            pltpu.VMEM((128, _EMBED), jnp.float32),
            pltpu.VMEM((_TAIL * _EMBED,), jnp.float32),
            pltpu.SemaphoreType.DMA,
            pltpu.SemaphoreType.DMA,
            pltpu.SemaphoreType.DMA,
            pltpu.SemaphoreType.DMA,
        ],
    )
    def k(wt_hbm, tail_hbm, out_hbm, in0, in1, r0, r1, c0v, c1v, tail_v,
          i0, i1, o0, o1):
        wid = lax.axis_index("s") * _NC + lax.axis_index("c")
        in_v = (in0, in1)
        rows_v = (r0, r1)
        comp_v = (c0v, c1v)
        isem = (i0, i1)
        osem = (o0, o1)
        iota = lax.iota(jnp.int32, 16)

        def c_of(m):
            return ((wid + m * _NW) % _VT_FULL) * 128

        def start_in(m, b):
            pltpu.async_copy(
                wt_hbm.at[:, pl.ds(c_of(m), 128)], in_v[b], isem[b])

        def wait_in(b):
            pltpu.make_async_copy(
                wt_hbm.at[:, pl.ds(0, 128)], in_v[b], isem[b]).wait()

        def start_out(m, b):
            pltpu.async_copy(
                comp_v[b], out_hbm.at[pl.ds(c_of(m), 128), :], osem[b])

        def wait_out(b):
            pltpu.make_async_copy(
                comp_v[0], out_hbm.at[pl.ds(0, 128), :], osem[b]).wait()

        def transpose(b):
            @plsc.parallel_loop(0, _EMBED * 8, unroll=4)
            def tr(eh):
                e = eh // 8
                h = eh % 8
                vec = in_v[b][e, pl.ds(h * 16, 16)]
                plsc.store_scatter(
                    rows_v[b], [iota + h * 16, iota * 0 + e], vec)

            @plsc.parallel_loop(0, 128, unroll=4)
            def rs(v):
                for half in range(2):
                    vec = rows_v[b][v, pl.ds(half * 16, 16)]
                    comp_v[b][v, pl.ds(half * 16, 16)] = vec

        @pl.when(wid == 0)
        def _():
            pltpu.sync_copy(tail_hbm, tail_v)
            for q in range(_TAIL * _EMBED // 16):
                c0v[q // 2, pl.ds((q % 2) * 16, 16)] = \
                    tail_v[pl.ds(q * 16, 16)]
            pltpu.sync_copy(
                c0v.at[pl.ds(0, _TAIL), :],
                out_hbm.at[pl.ds(_VT_FULL * 128, _TAIL), :])

        start_in(0, 0)
        start_in(1, 1)
        for b in range(2):
            wait_in(b)
            transpose(b)
            start_out(b, b)
            start_in(b + 2, b)

        def body(kk, carry):
            for b in range(2):
                m = 2 * kk + b
                wait_in(b)
                wait_out(b)
                transpose(b)
                start_out(m, b)
                start_in(m + 2, b)
            return carry

        lax.fori_loop(1, _NTW // 2, body, 0)
        for b in range(2):
            wait_in(b)
            wait_out(b)

    return k(wt, tail_lin)


@functools.partial(jax.jit, static_argnums=(2, 3))
def _sc_gather_flat(idx_flat, table, total, chunk):
    bpw = total // _NW
    nchunk = bpw // chunk

    @functools.partial(
        pl.kernel,
        mesh=_MESH,
        compiler_params=pltpu.CompilerParams(use_tc_tiling_on_sc=False),
        out_type=jax.ShapeDtypeStruct((total, _EMBED), jnp.float32),
        scratch_types=[
            pltpu.VMEM((bpw,), jnp.int32),
            pltpu.VMEM((2, chunk, _EMBED), jnp.float32),
            pltpu.SemaphoreType.DMA,
            pltpu.SemaphoreType.DMA,
            pltpu.SemaphoreType.DMA,
            pltpu.SemaphoreType.DMA,
        ],
    )
    def k(idx_hbm, table_hbm, out_hbm, idx_v, rows_v, g0, g1, w0, w1):
        wid = lax.axis_index("s") * _NC + lax.axis_index("c")
        base = wid * bpw
        gsem = (g0, g1)
        wsem = (w0, w1)

        pltpu.sync_copy(idx_hbm.at[pl.ds(base, bpw)], idx_v)

        gh = [None, None]
        wh = [None, None]

        def start_gather(i, b):
            gh[b] = pltpu.async_copy(
                table_hbm.at[idx_v.at[pl.ds(i * chunk, chunk)]],
                rows_v.at[b], gsem[b])

        start_gather(0, 0)
        if nchunk > 1:
            start_gather(1, 1)
        for i in range(nchunk):
            b = i % 2
            gh[b].wait()
            wh[b] = pltpu.async_copy(
                rows_v.at[b], out_hbm.at[pl.ds(base + i * chunk, chunk)],
                wsem[b])
            if i + 2 < nchunk:
                wh[b].wait()
                start_gather(i + 2, b)
        for b in range(min(2, nchunk)):
            if wh[b] is not None:
                wh[b].wait()

    return k(idx_flat, table)


@jax.jit
def _sc_gather_t(idx_flat, table):
    @functools.partial(
        pl.kernel,
        mesh=_MESH,
        compiler_params=pltpu.CompilerParams(
            use_tc_tiling_on_sc=False, needs_layout_passes=False),
        out_type=jax.ShapeDtypeStruct((_L, 4, 128, 8, 128), jnp.float32),
        scratch_types=[
            pltpu.VMEM((2, _BW), jnp.int32),
            pltpu.VMEM((2, _BW, _EMBED), jnp.float32),
            pltpu.VMEM((2, _BW, _P), jnp.float32),
            pltpu.VMEM((2, 4, 4, 8, 128), jnp.float32),
            pltpu.SemaphoreType.DMA,
            pltpu.SemaphoreType.DMA,
            pltpu.SemaphoreType.DMA,
            pltpu.SemaphoreType.DMA,
        ],
    )
    def k(idx_hbm, table_hbm, out_hbm, idx_v, rows_c, rows_v, obuf,
          g0, g1, w0, w1):
        wid = lax.axis_index("s") * _NC + lax.axis_index("c")
        b0 = wid * _BW
        gsem = (g0, g1)
        wsem = (w0, w1)
        iota = lax.iota(jnp.int32, 16)

        def load_and_gather(l, b):
            pltpu.sync_copy(idx_hbm.at[pl.ds(l * _B + b0, _BW)], idx_v.at[b])
            pltpu.async_copy(
                table_hbm.at[idx_v.at[b]], rows_c.at[b], gsem[b])

        def wait_gather(b):
            pltpu.make_async_copy(
                table_hbm.at[idx_v.at[b]], rows_c.at[b], gsem[b]).wait()

        def start_out(l, b):
            for r in range(4):
                pltpu.async_copy(
                    obuf.at[b, r], out_hbm.at[l, r, pl.ds(wid * 4, 4)],
                    wsem[b])

        def wait_out(b):
            for r in range(4):
                pltpu.make_async_copy(
                    obuf.at[0, 0], out_hbm.at[0, 0, pl.ds(0, 4)],
                    wsem[b]).wait()

        def transpose(b):
            @plsc.parallel_loop(0, _BW, unroll=4)
            def rs(v):
                for half in range(2):
                    vec = rows_c[b, v, pl.ds(half * 16, 16)]
                    plsc.store_scatter(
                        rows_v.at[b],
                        [iota * 0 + v, iota + half * 16], vec)

            @plsc.parallel_loop(0, 16, unroll=2)
            def tr(rt):
                r = rt // 4
                t = rt % 4
                for i in range(8):
                    col = iota * 0 + (r * 8 + i)
                    for h in range(8):
                        row = iota + (t * 128 + h * 16)
                        vec = plsc.load_gather(rows_v.at[b], [row, col])
                        obuf[b, r, t, i, pl.ds(h * 16, 16)] = vec

        load_and_gather(0, 0)
        load_and_gather(1, 1)
        for b in range(2):
            wait_gather(b)
            transpose(b)
            start_out(b, b)
            load_and_gather(b + 2, b)

        def body(kk, carry):
            for b in range(2):
                l = 2 * kk + b
                wait_gather(b)
                wait_out(b)
                transpose(b)
                start_out(l, b)

                @pl.when(l + 2 < _L)
                def _():
                    load_and_gather(l + 2, b)
            return carry

        lax.fori_loop(1, _L // 2, body, 0)
        for b in range(2):
            wait_out(b)

    return k(idx_flat, table)


def kernel(x, weight):
    wt = jnp.transpose(weight)
    tail_lin = weight[_VT_FULL * 128:].reshape(_TAIL * _EMBED)
    table = _detile_table(wt, tail_lin)
    idx = jnp.transpose(x).reshape(_L * _B).astype(jnp.int32)
    out = _sc_gather_flat(idx, table, _L * _B, 1280)
    return jnp.transpose(out.reshape(_L, _B, _EMBED), (1, 0, 2))


# speedup vs baseline: 1.4864x; 1.4864x over previous
import functools

import jax
import jax.numpy as jnp
from jax import lax
from jax.experimental import pallas as pl
from jax.experimental.pallas import tpu as pltpu
from jax.experimental.pallas import tpu_sc as plsc

_EMBED = 32
_NC = 2
_NS = 16
_NW = _NC * _NS


@functools.partial(jax.jit, static_argnums=(2, 3))
def _sc_gather(idx_flat, table, total, chunk):
    bpw = total // _NW
    nchunk = bpw // chunk
    mesh = plsc.VectorSubcoreMesh(core_axis_name="c", subcore_axis_name="s")

    @functools.partial(
        pl.kernel,
        mesh=mesh,
        compiler_params=pltpu.CompilerParams(use_tc_tiling_on_sc=False),
        out_type=jax.ShapeDtypeStruct((total, _EMBED), jnp.float32),
        scratch_types=[
            pltpu.VMEM((bpw,), jnp.int32),
            pltpu.VMEM((2, chunk, _EMBED), jnp.float32),
            pltpu.SemaphoreType.DMA,
            pltpu.SemaphoreType.DMA,
            pltpu.SemaphoreType.DMA,
            pltpu.SemaphoreType.DMA,
        ],
    )
    def k(idx_hbm, table_hbm, out_hbm, idx_v, rows_v, g0, g1, w0, w1):
        wid = lax.axis_index("s") * _NC + lax.axis_index("c")
        base = wid * bpw
        gsem = (g0, g1)
        wsem = (w0, w1)

        pltpu.sync_copy(idx_hbm.at[pl.ds(base, bpw)], idx_v)

        gh = [None, None]
        wh = [None, None]

        def start_gather(i, b):
            gh[b] = pltpu.async_copy(
                table_hbm.at[idx_v.at[pl.ds(i * chunk, chunk)]],
                rows_v.at[b], gsem[b])

        start_gather(0, 0)
        if nchunk > 1:
            start_gather(1, 1)
        for i in range(nchunk):
            b = i % 2
            gh[b].wait()
            wh[b] = pltpu.async_copy(
                rows_v.at[b], out_hbm.at[pl.ds(base + i * chunk, chunk)],
                wsem[b])
            if i + 2 < nchunk:
                wh[b].wait()
                start_gather(i + 2, b)
        for b in range(min(2, nchunk)):
            if wh[b] is not None:
                wh[b].wait()

    return k(idx_flat, table)


def kernel(x, weight):
    b, l = x.shape
    total = b * l
    flat = jnp.transpose(x).reshape(total).astype(jnp.int32)
    out = _sc_gather(flat, weight, total, 1280)
    return jnp.transpose(out.reshape(l, b, _EMBED), (1, 0, 2))
